# baseline (speedup 1.0000x reference)
import functools

import jax
import jax.numpy as jnp
from jax import lax
from jax.experimental import pallas as pl
from jax.experimental.pallas import tpu as pltpu
from jax.experimental.pallas import tpu_sc as plsc

DIM = 64
LANES = 16
NUM_CORES = 2
NUM_SUBCORES = 16
NUM_WORKERS = NUM_CORES * NUM_SUBCORES
NCOL = 7812
ITEM_BASE = 7816
NVCOL = ITEM_BASE + NCOL
STRIP = 489
NLOC = 123
NLOCPAD = NLOC + 21
RINGN = 3
W = 512
H = 3072
HPAD = H + 16
JUNK = 49279
TAIL_START = NCOL * 128


def _routing(user, pos, neg):
    a = jnp.concatenate([user, pos, neg]).astype(jnp.int32)
    n = a.shape[0]
    posn = jnp.arange(n, dtype=jnp.int32)
    item = (posn >= user.shape[0]).astype(jnp.int32)
    cid = a >> 7
    tail = cid >= NCOL
    vcol = jnp.where(tail, NVCOL + item, cid + ITEM_BASE * item)
    rloc = jnp.where(tail, a - TAIL_START, a & 127)
    rc = jnp.where(tail, rloc, ((vcol & 3) << 7) + rloc)
    order = jnp.argsort(vcol).astype(jnp.int32)
    vs = vcol[order]
    rc_s = rc[order]

    t32 = jnp.arange(NUM_WORKERS, dtype=jnp.int32)
    lo = jnp.searchsorted(vs, t32 * STRIP, side="left").astype(jnp.int32)
    hi_all = jnp.searchsorted(vs, (t32 + 1) * STRIP, side="left").astype(jnp.int32)
    ntail0 = jnp.searchsorted(vs, jnp.int32(NVCOL), side="left").astype(jnp.int32)
    hi_norm = jnp.minimum(hi_all, ntail0)

    g0 = (t32 * STRIP) >> 2
    j = jnp.arange(NLOCPAD, dtype=jnp.int32)
    qb = (g0[:, None] + j[None, :]) << 2
    gb = jnp.searchsorted(vs, qb, side="left").astype(jnp.int32)
    ge = jnp.searchsorted(vs, qb + 4, side="left").astype(jnp.int32)
    gb = jnp.clip(jnp.clip(gb, lo[:, None], hi_norm[:, None]) - lo[:, None], 0, H)
    ge = jnp.clip(jnp.clip(ge, lo[:, None], hi_norm[:, None]) - lo[:, None], 0, H)

    def seg(v):
        s = jnp.searchsorted(vs, v, side="left").astype(jnp.int32)
        e = jnp.searchsorted(vs, v + 1, side="left").astype(jnp.int32)
        s = jnp.clip(jnp.clip(s, lo, hi_all) - lo, 0, H)
        e = jnp.clip(jnp.clip(e, lo, hi_all) - lo, 0, H)
        return s, e

    s0, e0 = seg(jnp.int32(NVCOL))
    s1, e1 = seg(jnp.int32(NVCOL + 1))
    cnt = jnp.clip(hi_all - lo, 0, H)
    tbA = jnp.stack([s0, e0, s1, e1, cnt] + [cnt] * 11, axis=1)

    i = jnp.arange(HPAD, dtype=jnp.int32)
    src = jnp.clip(lo[:, None] + i[None, :], 0, n - 1)
    valid = i[None, :] < cnt[:, None]
    slotA = jnp.where(valid, order[src], JUNK)
    rcA = jnp.where(valid, rc_s[src], 0)
    return slotA, rcA, gb, ge, tbA


def _sc_extract(tt_u, tt_i, slotA, rcA, gbA, geA, tbA):
    mesh = plsc.VectorSubcoreMesh(core_axis_name="c", subcore_axis_name="s")

    @functools.partial(
        pl.kernel,
        mesh=mesh,
        out_type=jax.ShapeDtypeStruct((JUNK + 1, 2 * DIM), jnp.float32),
        compiler_params=pltpu.CompilerParams(use_tc_tiling_on_sc=True,
                                             needs_layout_passes=False),
        scratch_types=[
            pltpu.VMEM((RINGN, DIM, W), jnp.float32),
            pltpu.VMEM((2, LANES, 2 * DIM), jnp.float32),
            pltpu.VMEM((DIM, DIM), jnp.float32),
            pltpu.VMEM((DIM, DIM), jnp.float32),
            pltpu.VMEM((HPAD,), jnp.int32),
            pltpu.VMEM((HPAD,), jnp.int32),
            pltpu.VMEM((NLOCPAD,), jnp.int32),
            pltpu.VMEM((NLOCPAD,), jnp.int32),
            pltpu.VMEM((16,), jnp.int32),
            [pltpu.SemaphoreType.DMA] * RINGN,
            pltpu.SemaphoreType.DMA,
        ],
    )
    def k(ttu_hbm, tti_hbm, slot_hbm, rc_hbm, gb_hbm, ge_hbm, tb_hbm,
          stag_hbm, ring, ob, tlu, tli, slot_v, rc_v, gb_v, ge_v, tb_v,
          fsems, ssem):
        wid = lax.axis_index("s") * NUM_CORES + lax.axis_index("c")
        g0 = (wid * STRIP) >> 2

        pltpu.sync_copy(slot_hbm.at[wid], slot_v)
        pltpu.sync_copy(rc_hbm.at[wid], rc_v)
        pltpu.sync_copy(gb_hbm.at[wid], gb_v)
        pltpu.sync_copy(ge_hbm.at[wid], ge_v)
        pltpu.sync_copy(tb_hbm.at[wid], tb_v)

        iota = lax.iota(jnp.int32, LANES)

        def fetch(gg, r):
            fg = gg << 2
            is_item = fg >= ITEM_BASE
            col0 = jnp.minimum(jnp.where(is_item, fg - ITEM_BASE, fg),
                               NCOL - 4)
            cs = pl.multiple_of(col0 << 7, 128)

            @pl.when(is_item)
            def _():
                pltpu.async_copy(tti_hbm.at[:, pl.ds(cs, W)],
                                 ring.at[r], fsems[r])

            @pl.when(jnp.logical_not(is_item))
            def _():
                pltpu.async_copy(ttu_hbm.at[:, pl.ds(cs, W)],
                                 ring.at[r], fsems[r])

        def make_inner(ref, base, bound):
            def inner(it, gc):
                @pl.when(gc >= 2)
                def _():
                    pltpu.make_async_copy(stag_hbm.at[pl.ds(0, LANES)],
                                          ob.at[0], ssem).wait()
                h0 = base + it * LANES
                m = (h0 + iota) < bound
                rc16 = rc_v[pl.ds(h0, LANES)]
                sv16 = jnp.where(m, slot_v[pl.ds(h0, LANES)], JUNK)
                pb = gc & 1
                for d in range(DIM):
                    v = plsc.load_gather(
                        ref, [jnp.full((LANES,), d, jnp.int32), rc16])
                    plsc.store_scatter(
                        ob.at[pb], [iota, jnp.full((LANES,), d, jnp.int32)], v)
                pltpu.async_copy(ob.at[pb], stag_hbm.at[sv16], ssem)
                return gc + 1
            return inner

        for r in range(RINGN):
            fetch(g0 + r, r)

        def outer(o, gc):
            for r in range(RINGN):
                jj = o * RINGN + r
                pltpu.make_async_copy(ttu_hbm.at[:, pl.ds(0, W)],
                                      ring.at[r], fsems[r]).wait()
                pass

                @pl.when(jj + RINGN <= NLOC - 1)
                def _():
                    fetch(g0 + jj + RINGN, r)
            return gc

        gc = lax.fori_loop(0, NLOC // RINGN, outer, 0)

        for a in range(DIM // 8):
            pltpu.sync_copy(ttu_hbm.at[pl.ds(a * 8, 8),
                                       pl.ds(TAIL_START, DIM)],
                            tlu.at[pl.ds(a * 8, 8)])
            pltpu.sync_copy(tti_hbm.at[pl.ds(a * 8, 8),
                                       pl.ds(TAIL_START, DIM)],
                            tli.at[pl.ds(a * 8, 8)])
        tbv = tb_v[pl.ds(0, LANES)]
        nit0 = (tbv[1] - tbv[0] + LANES - 1) // LANES
        gc = lax.fori_loop(0, nit0, make_inner(tlu, tbv[0], tbv[1]), gc)
        nit1 = (tbv[3] - tbv[2] + LANES - 1) // LANES
        gc = lax.fori_loop(0, nit1, make_inner(tli, tbv[2], tbv[3]), gc)

        @pl.when(gc >= 1)
        def _():
            pltpu.make_async_copy(stag_hbm.at[pl.ds(0, LANES)], ob.at[0],
                                  ssem).wait()

        @pl.when(gc >= 2)
        def _():
            pltpu.make_async_copy(stag_hbm.at[pl.ds(0, LANES)], ob.at[0],
                                  ssem).wait()

    return k(tt_u, tt_i, slotA, rcA, gbA, geA, tbA)


def _tc_loss_body(u_ref, p_ref, n_ref, o_ref):
    u = u_ref[...][:, :DIM]
    p = p_ref[...][:, :DIM]
    nn = n_ref[...][:, :DIM]
    tmp = jnp.sum(u * (p - nn), axis=1)
    bpr = jnp.maximum(-tmp, 0.0) + jnp.log1p(jnp.exp(-jnp.abs(tmp)))

    @pl.when(pl.program_id(0) == 0)
    def _():
        o_ref[0, 0] = 0.0

    o_ref[0, 0] += jnp.sum(bpr)


def kernel(user, pos, neg, user_table, item_table):
    batch = user.shape[0]
    slotA, rcA, gbA, geA, tbA = _routing(
        user.astype(jnp.int32), pos.astype(jnp.int32), neg.astype(jnp.int32))
    staging = _sc_extract(user_table.T, item_table.T,
                          slotA, rcA, gbA, geA, tbA)
    blk = 1024
    nblk = batch // blk
    loss = pl.pallas_call(
        _tc_loss_body,
        grid=(nblk,),
        out_shape=jax.ShapeDtypeStruct((1, 1), jnp.float32),
        in_specs=[
            pl.BlockSpec((blk, 2 * DIM), lambda i: (i, 0)),
            pl.BlockSpec((blk, 2 * DIM), lambda i: (i + nblk, 0)),
            pl.BlockSpec((blk, 2 * DIM), lambda i: (i + 2 * nblk, 0)),
        ],
        out_specs=pl.BlockSpec((1, 1), lambda i: (0, 0),
                               memory_space=pltpu.SMEM),
    )(staging, staging, staging)
    return loss[0, 0]


# speedup vs baseline: 1.6736x; 1.6736x over previous
import functools

import jax
import jax.numpy as jnp
from jax import lax
from jax.experimental import pallas as pl
from jax.experimental.pallas import tpu as pltpu
from jax.experimental.pallas import tpu_sc as plsc

DIM = 64
LANES = 16
NUM_CORES = 2
NUM_SUBCORES = 16
NUM_WORKERS = NUM_CORES * NUM_SUBCORES
CHUNK = 128


def _sc_partial_scores(user_idx, pos_idx, neg_idx, user_table, item_table,
                       batch):
    b_per_w = batch // NUM_WORKERS
    n_chunks = b_per_w // CHUNK

    mesh = plsc.VectorSubcoreMesh(core_axis_name="c", subcore_axis_name="s")

    @functools.partial(
        pl.kernel,
        mesh=mesh,
        out_type=jax.ShapeDtypeStruct((batch, LANES), jnp.float32),
        compiler_params=pltpu.CompilerParams(use_tc_tiling_on_sc=False),
        scratch_types=[
            pltpu.VMEM((n_chunks, CHUNK), jnp.int32),
            pltpu.VMEM((n_chunks, CHUNK), jnp.int32),
            pltpu.VMEM((n_chunks, CHUNK), jnp.int32),
            pltpu.VMEM((b_per_w, DIM), jnp.float32),
            pltpu.VMEM((b_per_w, DIM), jnp.float32),
            pltpu.VMEM((b_per_w, DIM), jnp.float32),
            pltpu.VMEM((b_per_w, LANES), jnp.float32),
            pltpu.SemaphoreType.DMA,
        ],
    )
    def sc_kernel(user_hbm, pos_hbm, neg_hbm, ut_hbm, it_hbm, out_hbm,
                  idx_u, idx_p, idx_n, u_v, p_v, n_v, o_v, sem):
        wid = lax.axis_index("s") * NUM_CORES + lax.axis_index("c")
        base = wid * b_per_w

        copies = []
        for j in range(n_chunks):
            pltpu.sync_copy(user_hbm.at[wid, j], idx_u.at[j])
            pltpu.sync_copy(pos_hbm.at[wid, j], idx_p.at[j])
            pltpu.sync_copy(neg_hbm.at[wid, j], idx_n.at[j])
            dst = pl.ds(j * CHUNK, CHUNK)
            copies.append(pltpu.async_copy(ut_hbm.at[idx_u.at[j]],
                                           u_v.at[dst], sem))
            copies.append(pltpu.async_copy(it_hbm.at[idx_p.at[j]],
                                           p_v.at[dst], sem))
            copies.append(pltpu.async_copy(it_hbm.at[idx_n.at[j]],
                                           n_v.at[dst], sem))
        for c in copies:
            c.wait()

        def body(i, carry):
            acc = jnp.zeros((LANES,), jnp.float32)
            for c in range(DIM // LANES):
                sl = pl.ds(c * LANES, LANES)
                uu = u_v[i, sl]
                pp = p_v[i, sl]
                nn = n_v[i, sl]
                acc = acc + uu * (pp - nn)
            o_v[i, :] = acc
            return carry

        lax.fori_loop(0, b_per_w, body, 0)

        pltpu.sync_copy(o_v, out_hbm.at[pl.ds(base, b_per_w)])

    u3 = user_idx.reshape(NUM_WORKERS, n_chunks, CHUNK)
    p3 = pos_idx.reshape(NUM_WORKERS, n_chunks, CHUNK)
    n3 = neg_idx.reshape(NUM_WORKERS, n_chunks, CHUNK)
    return sc_kernel(u3, p3, n3, user_table, item_table)


def _tc_loss_body(w_ref, o_ref):
    w = w_ref[...]
    tmp = jnp.sum(w, axis=1)
    bpr = jnp.maximum(-tmp, 0.0) + jnp.log1p(jnp.exp(-jnp.abs(tmp)))
    o_ref[0, 0] = jnp.sum(bpr)


def kernel(user, pos, neg, user_table, item_table):
    batch = user.shape[0]
    partial = _sc_partial_scores(
        user.astype(jnp.int32), pos.astype(jnp.int32), neg.astype(jnp.int32),
        user_table, item_table, batch)
    loss = pl.pallas_call(
        _tc_loss_body,
        out_shape=jax.ShapeDtypeStruct((1, 1), jnp.float32),
        in_specs=[pl.BlockSpec(memory_space=pltpu.VMEM)],
        out_specs=pl.BlockSpec(memory_space=pltpu.SMEM),
    )(partial)
    return loss[0, 0]
